# baseline (speedup 1.0000x reference)
import functools

import numpy as np
import jax
import jax.numpy as jnp
from jax import lax
from jax.experimental import pallas as pl
from jax.experimental.pallas import tpu as pltpu
from jax.experimental.pallas import tpu_sc as plsc

_NUM_PATCHES = 16
_STALK = 64
_THRESHOLD = 0.5


def _edges():
    src, dst = [], []
    for i in range(_NUM_PATCHES):
        for j in range(i + 1, _NUM_PATCHES):
            src.extend([i, j])
            dst.extend([j, i])
    return np.array(src, dtype=np.int32), np.array(dst, dtype=np.int32)


_SRC, _DST = _edges()
_NE = _SRC.shape[0]
_NPAIR = _NE // 2
_NPAIR_PAD = 128
_NTRIPLES = 3360

_PAIR_IJ = np.zeros((_NPAIR_PAD, 16), dtype=np.int32)
_PAIR_IJ[:_NPAIR, 0] = _SRC[0::2]
_PAIR_IJ[:_NPAIR, 1] = _SRC[1::2]

_PAIR_MASK = ((_DST[:, None] == _DST[None, :])
              & (np.arange(_NE)[:, None] != np.arange(_NE)[None, :])
              ).astype(np.float32)

_HI = jax.lax.Precision.HIGHEST

_SC_MESH = plsc.VectorSubcoreMesh(core_axis_name="c", subcore_axis_name="s",
                                  num_cores=1)
_NW = 16
_PAIRS_PER_W = 8
_LANES = 16


def _cocycle_sc(rhoT_hbm, patches_hbm, ij_hbm, d2_hbm,
                rt_v, patches_v, ij_v, out_v):
    wid = lax.axis_index("s")
    pltpu.sync_copy(patches_hbm, patches_v)
    pltpu.sync_copy(ij_hbm, ij_v)

    for l in range(_PAIRS_PER_W):
        out_v[l, :] = jnp.zeros((_LANES,), jnp.float32)

    def pair_body(l, carry):
        p = wid * _PAIRS_PER_W + l
        b = lax.rem(l, 4)

        @pl.when(b == 0)
        def _stage():
            pltpu.sync_copy(rhoT_hbm.at[pl.ds(p * 2, 8)], rt_v)

        ij = ij_v[p, :]
        i = ij[0]
        j = ij[1]

        s0c = [patches_v[i, pl.ds(dc * 16, 16)] for dc in range(4)]
        s1c = [patches_v[j, pl.ds(dc * 16, 16)] for dc in range(4)]

        def q_body(q, y):
            idx = jnp.full((_LANES,), q, jnp.int32)
            y = list(y)
            for dc in range(4):
                s0 = s0c[dc].at[idx].get(mode='promise_in_bounds')
                s1 = s1c[dc].at[idx].get(mode='promise_in_bounds')
                d = dc * 16 + q
                for c in range(4):
                    y[c] = y[c] + rt_v[2 * b, d, pl.ds(c * 16, 16)] * s0
                    y[4 + c] = (y[4 + c]
                                + rt_v[2 * b + 1, d, pl.ds(c * 16, 16)] * s1)
            return tuple(y)

        y = lax.fori_loop(
            0, 16, q_body,
            tuple(jnp.zeros((_LANES,), jnp.float32) for _ in range(8)))
        acc = jnp.zeros((_LANES,), jnp.float32)
        for c in range(4):
            z = y[c] - y[4 + c]
            acc = acc + z * z
        out_v[l, :] = acc
        return carry

    @pl.when(wid < _NPAIR // _PAIRS_PER_W)
    def _():
        lax.fori_loop(0, _PAIRS_PER_W, pair_body, 0)

    pltpu.sync_copy(out_v, d2_hbm.at[pl.ds(wid * _PAIRS_PER_W, _PAIRS_PER_W)])


@functools.partial(
    pl.kernel,
    out_type=jax.ShapeDtypeStruct((_NPAIR_PAD, _LANES), jnp.float32),
    mesh=_SC_MESH,
    scratch_types=[
        pltpu.VMEM((8, _STALK, _STALK), jnp.float32),
        pltpu.VMEM((_NUM_PATCHES, _STALK), jnp.float32),
        pltpu.VMEM((_NPAIR_PAD, _LANES), jnp.int32),
        pltpu.VMEM((_PAIRS_PER_W, _LANES), jnp.float32),
    ],
)
def _cocycle_sc_call(rhoT, patches, ij, d2_out, rt_v, patches_v, ij_v, out_v):
    _cocycle_sc(rhoT, patches, ij, d2_out, rt_v, patches_v, ij_v, out_v)


def _transpose_tc_kernel(rho_ref, rhoT_ref):
    rhoT_ref[...] = jnp.swapaxes(rho_ref[...], 1, 2)


def _dense_tc_kernel(patches_ref, rho2_ref, w_ref, mask_ref,
                     comp_ref, gsec_ref):
    patches = patches_ref[...]
    rho2 = rho2_ref[...]

    g = jax.lax.dot_general(
        rho2, rho2,
        dimension_numbers=(((1,), (1,)), ((), ())), precision=_HI)
    rr = lax.broadcasted_iota(jnp.int32, (_NE, _NE), 0)
    cc = lax.broadcasted_iota(jnp.int32, (_NE, _NE), 1)
    eye = (rr == cc).astype(jnp.float32)
    n_row = jnp.sum(g * eye, axis=1, keepdims=True)
    n_col = jnp.sum(g * eye, axis=0, keepdims=True)
    v2 = jnp.maximum(n_row + n_col - 2.0 * g, 0.0)
    comp_ref[...] = jnp.broadcast_to(
        jnp.sum(jnp.sqrt(v2) * mask_ref[...]) / _NTRIPLES, (1, 1))

    m = jnp.sum(patches, axis=0, keepdims=True) / _NUM_PATCHES
    gsec_ref[...] = jax.lax.dot_general(
        m, w_ref[...],
        dimension_numbers=(((1,), (1,)), ((), ())), precision=_HI)


def _defect_epilogue_kernel(d2_ref, defects_ref, scalars_ref):
    d2 = jnp.sum(d2_ref[...], axis=-1, keepdims=True)[0:_NPAIR]
    dv = jnp.sqrt(d2)
    defects_ref[...] = jnp.broadcast_to(dv, (_NPAIR, 2))
    max_defect = jnp.max(dv)
    mean_defect = jnp.sum(dv) / _NPAIR
    consistency = jnp.exp(-mean_defect / _THRESHOLD)
    scalars_ref[...] = jnp.concatenate(
        [jnp.broadcast_to(v, (1, 1)) for v in
         (max_defect, mean_defect, consistency)], axis=1)


def kernel(patches, restriction_maps, W):
    patches = patches.astype(jnp.float32)
    rho3 = restriction_maps.astype(jnp.float32)
    mask = jnp.asarray(_PAIR_MASK)

    rhoT = pl.pallas_call(
        _transpose_tc_kernel,
        grid=(15,),
        in_specs=[pl.BlockSpec((16, _STALK, _STALK), lambda g: (g, 0, 0))],
        out_specs=pl.BlockSpec((16, _STALK, _STALK), lambda g: (g, 0, 0)),
        out_shape=jax.ShapeDtypeStruct((_NE, _STALK, _STALK), jnp.float32),
    )(rho3)
    rho2 = rhoT.reshape(_NE, _STALK * _STALK)

    d2 = _cocycle_sc_call(rhoT, patches, jnp.asarray(_PAIR_IJ))

    comp, gsec = pl.pallas_call(
        _dense_tc_kernel,
        out_shape=(
            jax.ShapeDtypeStruct((1, 1), jnp.float32),
            jax.ShapeDtypeStruct((1, _STALK), jnp.float32),
        ),
    )(patches, rho2, W.astype(jnp.float32), mask)

    defects2, scalars = pl.pallas_call(
        _defect_epilogue_kernel,
        out_shape=(
            jax.ShapeDtypeStruct((_NPAIR, 2), jnp.float32),
            jax.ShapeDtypeStruct((1, 3), jnp.float32),
        ),
    )(d2)

    defects = defects2.reshape(_NE)
    max_defect = scalars[0, 0]
    mean_defect = scalars[0, 1]
    consistency = scalars[0, 2]
    comp_defect = comp[0, 0]
    global_section = gsec.reshape(_STALK)
    gluing_satisfied = max_defect < _THRESHOLD
    return (defects, max_defect, mean_defect, consistency, comp_defect,
            global_section, gluing_satisfied)


# speedup vs baseline: 1.0809x; 1.0809x over previous
import functools

import numpy as np
import jax
import jax.numpy as jnp
from jax import lax
from jax.experimental import pallas as pl
from jax.experimental.pallas import tpu as pltpu
from jax.experimental.pallas import tpu_sc as plsc

_NUM_PATCHES = 16
_STALK = 64
_THRESHOLD = 0.5


def _edges():
    src, dst = [], []
    for i in range(_NUM_PATCHES):
        for j in range(i + 1, _NUM_PATCHES):
            src.extend([i, j])
            dst.extend([j, i])
    return np.array(src, dtype=np.int32), np.array(dst, dtype=np.int32)


_SRC, _DST = _edges()
_NE = _SRC.shape[0]
_NPAIR = _NE // 2
_NPAIR_PAD = 128
_NTRIPLES = 3360

_PAIR_IJ = np.zeros((_NPAIR_PAD, 16), dtype=np.int32)
_PAIR_IJ[:_NPAIR, 0] = _SRC[0::2]
_PAIR_IJ[:_NPAIR, 1] = _SRC[1::2]

_PAIR_MASK = ((_DST[:, None] == _DST[None, :])
              & (np.arange(_NE)[:, None] != np.arange(_NE)[None, :])
              ).astype(np.float32)

_HI = jax.lax.Precision.HIGHEST

_SC_MESH = plsc.VectorSubcoreMesh(core_axis_name="c", subcore_axis_name="s",
                                  num_cores=1)
_NW = 16
_PAIRS_PER_W = 8
_LANES = 16


def _cocycle_sc(rhoT_hbm, patches_hbm, ij_hbm, d2_hbm,
                rt_v, patches_v, ij_v, out_v):
    wid = lax.axis_index("s")
    pltpu.sync_copy(patches_hbm, patches_v)
    pltpu.sync_copy(ij_hbm, ij_v)

    for l in range(_PAIRS_PER_W):
        out_v[l, :] = jnp.zeros((_LANES,), jnp.float32)

    def pair_body(l, carry):
        p = wid * _PAIRS_PER_W + l
        b = lax.rem(l, 4)

        @pl.when(b == 0)
        def _stage():
            pltpu.sync_copy(rhoT_hbm.at[pl.ds(p * 2, 8)], rt_v)

        ij = ij_v[p, :]
        i = ij[0]
        j = ij[1]

        s0c = [patches_v[i, pl.ds(dc * 16, 16)] for dc in range(4)]
        s1c = [patches_v[j, pl.ds(dc * 16, 16)] for dc in range(4)]

        def q_body(q, y):
            idx = jnp.full((_LANES,), q, jnp.int32)
            y = list(y)
            for dc in range(4):
                s0 = s0c[dc].at[idx].get(mode='promise_in_bounds')
                s1 = s1c[dc].at[idx].get(mode='promise_in_bounds')
                d = dc * 16 + q
                for c in range(4):
                    y[c] = y[c] + rt_v[2 * b, d, pl.ds(c * 16, 16)] * s0
                    y[4 + c] = (y[4 + c]
                                + rt_v[2 * b + 1, d, pl.ds(c * 16, 16)] * s1)
            return tuple(y)

        y = lax.fori_loop(
            0, 16, q_body,
            tuple(jnp.zeros((_LANES,), jnp.float32) for _ in range(8)))
        acc = jnp.zeros((_LANES,), jnp.float32)
        for c in range(4):
            z = y[c] - y[4 + c]
            acc = acc + z * z
        out_v[l, :] = acc
        return carry

    @pl.when(wid < _NPAIR // _PAIRS_PER_W)
    def _():
        lax.fori_loop(0, _PAIRS_PER_W, pair_body, 0)

    pltpu.sync_copy(out_v, d2_hbm.at[pl.ds(wid * _PAIRS_PER_W, _PAIRS_PER_W)])


@functools.partial(
    pl.kernel,
    out_type=jax.ShapeDtypeStruct((_NPAIR_PAD, _LANES), jnp.float32),
    mesh=_SC_MESH,
    scratch_types=[
        pltpu.VMEM((8, _STALK, _STALK), jnp.float32),
        pltpu.VMEM((_NUM_PATCHES, _STALK), jnp.float32),
        pltpu.VMEM((_NPAIR_PAD, _LANES), jnp.int32),
        pltpu.VMEM((_PAIRS_PER_W, _LANES), jnp.float32),
    ],
)
def _cocycle_sc_call(rhoT, patches, ij, d2_out, rt_v, patches_v, ij_v, out_v):
    _cocycle_sc(rhoT, patches, ij, d2_out, rt_v, patches_v, ij_v, out_v)


def _transpose_tc_kernel(rho_ref, rhoT_ref):
    rhoT_ref[...] = jnp.swapaxes(rho_ref[...], 1, 2)


def _dense_tc_kernel(patches_ref, rho2_ref, w_ref, mask_ref,
                     comp_ref, gsec_ref):
    patches = patches_ref[...]
    rho2 = rho2_ref[...]

    g = jax.lax.dot_general(
        rho2, rho2,
        dimension_numbers=(((1,), (1,)), ((), ())), precision=_HI)
    rr = lax.broadcasted_iota(jnp.int32, (_NE, _NE), 0)
    cc = lax.broadcasted_iota(jnp.int32, (_NE, _NE), 1)
    eye = (rr == cc).astype(jnp.float32)
    n_row = jnp.sum(g * eye, axis=1, keepdims=True)
    n_col = jnp.sum(g * eye, axis=0, keepdims=True)
    v2 = jnp.maximum(n_row + n_col - 2.0 * g, 0.0)
    comp_ref[...] = jnp.broadcast_to(
        jnp.sum(jnp.sqrt(v2) * mask_ref[...]) / _NTRIPLES, (1, 1))

    m = jnp.sum(patches, axis=0, keepdims=True) / _NUM_PATCHES
    gsec_ref[...] = jax.lax.dot_general(
        m, w_ref[...],
        dimension_numbers=(((1,), (1,)), ((), ())), precision=_HI)


def _defect_epilogue_kernel(d2_ref, defects_ref, scalars_ref):
    d2 = jnp.sum(d2_ref[...], axis=-1, keepdims=True)[0:_NPAIR]
    dv = jnp.sqrt(d2)
    defects_ref[...] = jnp.broadcast_to(dv, (_NPAIR, 2))
    max_defect = jnp.max(dv)
    mean_defect = jnp.sum(dv) / _NPAIR
    consistency = jnp.exp(-mean_defect / _THRESHOLD)
    scalars_ref[...] = jnp.concatenate(
        [jnp.broadcast_to(v, (1, 1)) for v in
         (max_defect, mean_defect, consistency)], axis=1)


def kernel(patches, restriction_maps, W):
    patches = patches.astype(jnp.float32)
    rho3 = restriction_maps.astype(jnp.float32)
    mask = jnp.asarray(_PAIR_MASK)

    rhoT = pl.pallas_call(
        _transpose_tc_kernel,
        out_shape=jax.ShapeDtypeStruct((_NE, _STALK, _STALK), jnp.float32),
    )(rho3)
    rho2 = rhoT.reshape(_NE, _STALK * _STALK)

    d2 = _cocycle_sc_call(rhoT, patches, jnp.asarray(_PAIR_IJ))

    comp, gsec = pl.pallas_call(
        _dense_tc_kernel,
        out_shape=(
            jax.ShapeDtypeStruct((1, 1), jnp.float32),
            jax.ShapeDtypeStruct((1, _STALK), jnp.float32),
        ),
    )(patches, rho2, W.astype(jnp.float32), mask)

    defects2, scalars = pl.pallas_call(
        _defect_epilogue_kernel,
        out_shape=(
            jax.ShapeDtypeStruct((_NPAIR, 2), jnp.float32),
            jax.ShapeDtypeStruct((1, 3), jnp.float32),
        ),
    )(d2)

    defects = defects2.reshape(_NE)
    max_defect = scalars[0, 0]
    mean_defect = scalars[0, 1]
    consistency = scalars[0, 2]
    comp_defect = comp[0, 0]
    global_section = gsec.reshape(_STALK)
    gluing_satisfied = max_defect < _THRESHOLD
    return (defects, max_defect, mean_defect, consistency, comp_defect,
            global_section, gluing_satisfied)


# speedup vs baseline: 1.2167x; 1.1257x over previous
import functools

import numpy as np
import jax
import jax.numpy as jnp
from jax import lax
from jax.experimental import pallas as pl
from jax.experimental.pallas import tpu as pltpu
from jax.experimental.pallas import tpu_sc as plsc

_NUM_PATCHES = 16
_STALK = 64
_THRESHOLD = 0.5


def _edges():
    src, dst = [], []
    for i in range(_NUM_PATCHES):
        for j in range(i + 1, _NUM_PATCHES):
            src.extend([i, j])
            dst.extend([j, i])
    return np.array(src, dtype=np.int32), np.array(dst, dtype=np.int32)


_SRC, _DST = _edges()
_NE = _SRC.shape[0]
_NPAIR = _NE // 2
_NPAIR_PAD = 128
_NTRIPLES = 3360

_PAIR_IJ = np.zeros((_NPAIR_PAD, 16), dtype=np.int32)
_PAIR_IJ[:_NPAIR, 0] = _SRC[0::2]
_PAIR_IJ[:_NPAIR, 1] = _SRC[1::2]

_PAIR_MASK = ((_DST[:, None] == _DST[None, :])
              & (np.arange(_NE)[:, None] != np.arange(_NE)[None, :])
              ).astype(np.float32)

_HI = jax.lax.Precision.HIGHEST

_SC_MESH = plsc.VectorSubcoreMesh(core_axis_name="c", subcore_axis_name="s",
                                  num_cores=1)
_NW = 16
_PAIRS_PER_W = 8
_LANES = 16


def _cocycle_sc(rhoT_hbm, patches_hbm, ij_hbm, d2_hbm,
                rt_v, patches_v, ij_v, out_v):
    wid = lax.axis_index("s")
    pltpu.sync_copy(patches_hbm, patches_v)
    pltpu.sync_copy(ij_hbm, ij_v)

    for l in range(_PAIRS_PER_W):
        out_v[l, :] = jnp.zeros((_LANES,), jnp.float32)

    def pair_body(l, carry):
        p = wid * _PAIRS_PER_W + l
        b = lax.rem(l, 4)

        @pl.when(b == 0)
        def _stage():
            pltpu.sync_copy(rhoT_hbm.at[pl.ds(p * 2, 8)], rt_v)

        ij = ij_v[p, :]
        i = ij[0]
        j = ij[1]

        s0c = [patches_v[i, pl.ds(dc * 16, 16)] for dc in range(4)]
        s1c = [patches_v[j, pl.ds(dc * 16, 16)] for dc in range(4)]

        def q_body(q, y):
            idx = jnp.full((_LANES,), q, jnp.int32)
            y = list(y)
            for dc in range(4):
                s0 = s0c[dc].at[idx].get(mode='promise_in_bounds')
                s1 = s1c[dc].at[idx].get(mode='promise_in_bounds')
                d = dc * 16 + q
                for c in range(4):
                    y[c] = y[c] + rt_v[2 * b, d, pl.ds(c * 16, 16)] * s0
                    y[4 + c] = (y[4 + c]
                                + rt_v[2 * b + 1, d, pl.ds(c * 16, 16)] * s1)
            return tuple(y)

        y = lax.fori_loop(
            0, 16, q_body,
            tuple(jnp.zeros((_LANES,), jnp.float32) for _ in range(8)))
        acc = jnp.zeros((_LANES,), jnp.float32)
        for c in range(4):
            z = y[c] - y[4 + c]
            acc = acc + z * z
        out_v[l, :] = acc
        return carry

    @pl.when(wid < _NPAIR // _PAIRS_PER_W)
    def _():
        lax.fori_loop(0, _PAIRS_PER_W, pair_body, 0)

    pltpu.sync_copy(out_v, d2_hbm.at[pl.ds(wid * _PAIRS_PER_W, _PAIRS_PER_W)])


@functools.partial(
    pl.kernel,
    out_type=jax.ShapeDtypeStruct((_NPAIR_PAD, _LANES), jnp.float32),
    mesh=_SC_MESH,
    scratch_types=[
        pltpu.VMEM((8, _STALK, _STALK), jnp.float32),
        pltpu.VMEM((_NUM_PATCHES, _STALK), jnp.float32),
        pltpu.VMEM((_NPAIR_PAD, _LANES), jnp.int32),
        pltpu.VMEM((_PAIRS_PER_W, _LANES), jnp.float32),
    ],
)
def _cocycle_sc_call(rhoT, patches, ij, d2_out, rt_v, patches_v, ij_v, out_v):
    _cocycle_sc(rhoT, patches, ij, d2_out, rt_v, patches_v, ij_v, out_v)


def _dense_tc_kernel(patches_ref, rho2_ref, w_ref, mask_ref,
                     comp_ref, gsec_ref):
    patches = patches_ref[...]
    rho2 = rho2_ref[...]

    g = jax.lax.dot_general(
        rho2, rho2,
        dimension_numbers=(((1,), (1,)), ((), ())), precision=_HI)
    rr = lax.broadcasted_iota(jnp.int32, (_NE, _NE), 0)
    cc = lax.broadcasted_iota(jnp.int32, (_NE, _NE), 1)
    eye = (rr == cc).astype(jnp.float32)
    n_row = jnp.sum(g * eye, axis=1, keepdims=True)
    n_col = jnp.sum(g * eye, axis=0, keepdims=True)
    v2 = jnp.maximum(n_row + n_col - 2.0 * g, 0.0)
    comp_ref[...] = jnp.broadcast_to(
        jnp.sum(jnp.sqrt(v2) * mask_ref[...]) / _NTRIPLES, (1, 1))

    m = jnp.sum(patches, axis=0, keepdims=True) / _NUM_PATCHES
    gsec_ref[...] = jax.lax.dot_general(
        m, w_ref[...],
        dimension_numbers=(((1,), (1,)), ((), ())), precision=_HI)


def _defect_epilogue_kernel(d2_ref, defects_ref, scalars_ref):
    d2 = jnp.sum(d2_ref[...], axis=-1, keepdims=True)[0:_NPAIR]
    dv = jnp.sqrt(d2)
    defects_ref[...] = jnp.broadcast_to(dv, (_NPAIR, 2))
    max_defect = jnp.max(dv)
    mean_defect = jnp.sum(dv) / _NPAIR
    consistency = jnp.exp(-mean_defect / _THRESHOLD)
    scalars_ref[...] = jnp.concatenate(
        [jnp.broadcast_to(v, (1, 1)) for v in
         (max_defect, mean_defect, consistency)], axis=1)


def kernel(patches, restriction_maps, W):
    patches = patches.astype(jnp.float32)
    rho3 = restriction_maps.astype(jnp.float32)
    mask = jnp.asarray(_PAIR_MASK)

    rhoT = jnp.swapaxes(rho3, 1, 2)
    rho2 = rhoT.reshape(_NE, _STALK * _STALK)

    d2 = _cocycle_sc_call(rhoT, patches, jnp.asarray(_PAIR_IJ))

    comp, gsec = pl.pallas_call(
        _dense_tc_kernel,
        out_shape=(
            jax.ShapeDtypeStruct((1, 1), jnp.float32),
            jax.ShapeDtypeStruct((1, _STALK), jnp.float32),
        ),
    )(patches, rho2, W.astype(jnp.float32), mask)

    defects2, scalars = pl.pallas_call(
        _defect_epilogue_kernel,
        out_shape=(
            jax.ShapeDtypeStruct((_NPAIR, 2), jnp.float32),
            jax.ShapeDtypeStruct((1, 3), jnp.float32),
        ),
    )(d2)

    defects = defects2.reshape(_NE)
    max_defect = scalars[0, 0]
    mean_defect = scalars[0, 1]
    consistency = scalars[0, 2]
    comp_defect = comp[0, 0]
    global_section = gsec.reshape(_STALK)
    gluing_satisfied = max_defect < _THRESHOLD
    return (defects, max_defect, mean_defect, consistency, comp_defect,
            global_section, gluing_satisfied)
